# Initial kernel scaffold; baseline (speedup 1.0000x reference)
import jax
import jax.numpy as jnp
from jax.experimental import pallas as pl


def kernel(weights):
    raise NotImplementedError("write your pallas kernel here")


# speedup vs baseline: 22.5051x; 22.5051x over previous
import functools
import math

import jax
import jax.numpy as jnp
from jax import lax
from jax.experimental import pallas as pl
from jax.experimental.pallas import tpu as pltpu
from jax.experimental.pallas import tpu_sc as plsc


def kernel(weights):
    B, V, U = weights.shape
    FIRST = math.floor(V / math.e - 1.0) + 1
    NSTEP = V - FIRST
    K = 35
    NB = NSTEP // K
    assert K * NB == NSTEP and NB % 2 == 1 and U % 16 == 0 and V % 16 == 0
    NCHUNK = U // 16
    L = 16

    mesh = plsc.VectorSubcoreMesh(core_axis_name="c", subcore_axis_name="s")

    @functools.partial(
        pl.kernel,
        mesh=mesh,
        compiler_params=pltpu.CompilerParams(
            use_tc_tiling_on_sc=False, needs_layout_passes=False
        ),
        out_type=(
            jax.ShapeDtypeStruct((B, V), jnp.int32),
            jax.ShapeDtypeStruct((B, L), jnp.float32),
        ),
        scratch_types=[
            pltpu.VMEM((K, U), jnp.float32),
            pltpu.VMEM((K, U), jnp.float32),
            pltpu.VMEM((U,), jnp.float32),
            pltpu.VMEM((V,), jnp.int32),
            pltpu.VMEM((L,), jnp.float32),
            pltpu.SemaphoreType.DMA,
            pltpu.SemaphoreType.DMA,
        ],
    )
    def greedy(w_hbm, seq_hbm, nsz_hbm, buf0, buf1, madd, seq, szv, sem0, sem1):
        wid = lax.axis_index("s") * 2 + lax.axis_index("c")

        @pl.when(wid < B)
        def _body():
            b = wid
            lanes = lax.iota(jnp.int32, L)
            lane0 = lanes == 0
            zf = jnp.zeros((L,), jnp.float32)
            zi = jnp.zeros((L,), jnp.int32)
            for c in range(NCHUNK):
                madd[pl.ds(c * L, L)] = zf
            for c in range(V // L):
                seq[pl.ds(c * L, L)] = zi
            pltpu.async_copy(w_hbm.at[b, pl.ds(FIRST, K), :], buf0, sem0)
            pltpu.async_copy(w_hbm.at[b, pl.ds(FIRST + K, K), :], buf1, sem1)

            def do_block(g, size, buf, sem):
                pltpu.make_async_copy(
                    w_hbm.at[b, pl.ds(FIRST, K), :], buf, sem
                ).wait()

                def row(r, sz):
                    i = FIRST + g * K + r
                    best_v = jnp.full((L,), -jnp.inf, jnp.float32)
                    best_i = zi
                    for c in range(NCHUNK):
                        v = buf[r, pl.ds(c * L, L)] + madd[pl.ds(c * L, L)]
                        if c == 0:
                            v = jnp.where(lane0, jnp.float32(0.0), v)
                        idx = lanes + jnp.int32(c * L)
                        upd = v > best_v
                        best_i = jnp.where(upd, idx, best_i)
                        best_v = jnp.where(upd, v, best_v)
                    m = jnp.max(best_v)
                    s = jnp.min(
                        jnp.where(best_v == m, best_i, jnp.int32(1 << 30))
                    )
                    plsc.store_scatter(
                        seq,
                        [jnp.full((L,), i, jnp.int32)],
                        jnp.full((L,), s, jnp.int32),
                        mask=lane0,
                    )
                    plsc.store_scatter(
                        madd,
                        [jnp.full((L,), s, jnp.int32)],
                        jnp.full((L,), jnp.float32(-1e9)),
                        mask=lane0 & (s != 0),
                    )
                    return sz + m

                return lax.fori_loop(0, K, row, size)

            def pair(g2, size):
                g0 = 2 * g2
                size = do_block(g0, size, buf0, sem0)

                @pl.when(g0 + 2 < NB)
                def _():
                    pltpu.async_copy(
                        w_hbm.at[b, pl.ds(FIRST + (g0 + 2) * K, K), :],
                        buf0,
                        sem0,
                    )

                size = do_block(g0 + 1, size, buf1, sem1)

                @pl.when(g0 + 3 < NB)
                def _():
                    pltpu.async_copy(
                        w_hbm.at[b, pl.ds(FIRST + (g0 + 3) * K, K), :],
                        buf1,
                        sem1,
                    )

                return size

            size = lax.fori_loop(0, NB // 2, pair, jnp.float32(0.0))
            size = do_block(jnp.int32(NB - 1), size, buf0, sem0)
            szv[...] = jnp.full((L,), jnp.float32(0.0)) - size
            pltpu.sync_copy(seq, seq_hbm.at[b, :])
            pltpu.sync_copy(szv, nsz_hbm.at[b, :])

    seqs, nsz = greedy(weights)
    return nsz[:, 0], seqs


# speedup vs baseline: 32.6408x; 1.4504x over previous
import functools
import math

import jax
import jax.numpy as jnp
from jax import lax
from jax.experimental import pallas as pl
from jax.experimental.pallas import tpu as pltpu
from jax.experimental.pallas import tpu_sc as plsc


def kernel(weights):
    B, V, U = weights.shape
    FIRST = math.floor(V / math.e - 1.0) + 1
    NSTEP = V - FIRST
    K = 35
    NB = NSTEP // K
    assert K * NB == NSTEP and NB % 2 == 1 and U % 16 == 0 and V % 16 == 0
    NCHUNK = U // 16
    L = 16

    mesh = plsc.VectorSubcoreMesh(core_axis_name="c", subcore_axis_name="s")

    @functools.partial(
        pl.kernel,
        mesh=mesh,
        compiler_params=pltpu.CompilerParams(
            use_tc_tiling_on_sc=False, needs_layout_passes=False
        ),
        out_type=(
            jax.ShapeDtypeStruct((B, V), jnp.int32),
            jax.ShapeDtypeStruct((B, L), jnp.float32),
        ),
        scratch_types=[
            pltpu.VMEM((K, U), jnp.float32),
            pltpu.VMEM((K, U), jnp.float32),
            pltpu.VMEM((U,), jnp.float32),
            pltpu.VMEM((V,), jnp.int32),
            pltpu.VMEM((L,), jnp.float32),
            pltpu.SemaphoreType.DMA,
            pltpu.SemaphoreType.DMA,
        ],
    )
    def greedy(w_hbm, seq_hbm, nsz_hbm, buf0, buf1, madd, seq, szv, sem0, sem1):
        wid = lax.axis_index("s") * 2 + lax.axis_index("c")

        @pl.when(wid < B)
        def _body():
            b = wid
            lanes = lax.iota(jnp.int32, L)
            lane0 = lanes == 0
            zf = jnp.zeros((L,), jnp.float32)
            zi = jnp.zeros((L,), jnp.int32)
            for c in range(NCHUNK):
                madd[pl.ds(c * L, L)] = zf
            for c in range(V // L):
                seq[pl.ds(c * L, L)] = zi
            pltpu.async_copy(w_hbm.at[b, pl.ds(FIRST, K), :], buf0, sem0)
            pltpu.async_copy(w_hbm.at[b, pl.ds(FIRST + K, K), :], buf1, sem1)

            def do_block(g, size, buf, sem):
                pltpu.make_async_copy(
                    w_hbm.at[b, pl.ds(FIRST, K), :], buf, sem
                ).wait()

                def row(r, sz):
                    i = FIRST + g * K + r
                    NACC = 4
                    best_v = [jnp.full((L,), -jnp.inf, jnp.float32)] * NACC
                    best_i = [zi] * NACC
                    for c in range(NCHUNK):
                        a = c % NACC
                        v = buf[r, pl.ds(c * L, L)] + madd[pl.ds(c * L, L)]
                        if c == 0:
                            v = jnp.where(lane0, jnp.float32(0.0), v)
                        idx = lanes + jnp.int32(c * L)
                        upd = v > best_v[a]
                        best_i[a] = jnp.where(upd, idx, best_i[a])
                        best_v[a] = jnp.where(upd, v, best_v[a])
                    while len(best_v) > 1:
                        va, vb = best_v[0], best_v[1]
                        ia, ib = best_i[0], best_i[1]
                        take_b = (vb > va) | ((vb == va) & (ib < ia))
                        best_v = best_v[2:] + [jnp.where(take_b, vb, va)]
                        best_i = best_i[2:] + [jnp.where(take_b, ib, ia)]
                    m = jnp.max(best_v[0])
                    s = jnp.min(
                        jnp.where(best_v[0] == m, best_i[0], jnp.int32(1 << 30))
                    )
                    plsc.store_scatter(
                        seq,
                        [jnp.full((L,), i, jnp.int32)],
                        jnp.full((L,), s, jnp.int32),
                        mask=lane0,
                    )
                    plsc.store_scatter(
                        madd,
                        [jnp.full((L,), s, jnp.int32)],
                        jnp.full((L,), jnp.float32(-1e9)),
                        mask=lane0 & (s != 0),
                    )
                    return sz + m

                return lax.fori_loop(0, K, row, size)

            def pair(g2, size):
                g0 = 2 * g2
                size = do_block(g0, size, buf0, sem0)

                @pl.when(g0 + 2 < NB)
                def _():
                    pltpu.async_copy(
                        w_hbm.at[b, pl.ds(FIRST + (g0 + 2) * K, K), :],
                        buf0,
                        sem0,
                    )

                size = do_block(g0 + 1, size, buf1, sem1)

                @pl.when(g0 + 3 < NB)
                def _():
                    pltpu.async_copy(
                        w_hbm.at[b, pl.ds(FIRST + (g0 + 3) * K, K), :],
                        buf1,
                        sem1,
                    )

                return size

            size = lax.fori_loop(0, NB // 2, pair, jnp.float32(0.0))
            size = do_block(jnp.int32(NB - 1), size, buf0, sem0)
            szv[...] = jnp.full((L,), jnp.float32(0.0)) - size
            pltpu.sync_copy(seq, seq_hbm.at[b, :])
            pltpu.sync_copy(szv, nsz_hbm.at[b, :])

    seqs, nsz = greedy(weights)
    return nsz[:, 0], seqs


# speedup vs baseline: 32.7396x; 1.0030x over previous
import functools
import math

import jax
import jax.numpy as jnp
from jax import lax
from jax.experimental import pallas as pl
from jax.experimental.pallas import tpu as pltpu
from jax.experimental.pallas import tpu_sc as plsc


def kernel(weights):
    B, V, U = weights.shape
    FIRST = math.floor(V / math.e - 1.0) + 1
    NSTEP = V - FIRST
    K = 35
    NB = NSTEP // K
    assert K * NB == NSTEP and NB % 2 == 1 and U % 16 == 0 and V % 16 == 0
    NCHUNK = U // 16
    L = 16

    mesh = plsc.VectorSubcoreMesh(core_axis_name="c", subcore_axis_name="s")

    @functools.partial(
        pl.kernel,
        mesh=mesh,
        compiler_params=pltpu.CompilerParams(
            use_tc_tiling_on_sc=False, needs_layout_passes=False
        ),
        out_type=(
            jax.ShapeDtypeStruct((B, V), jnp.int32),
            jax.ShapeDtypeStruct((B, L), jnp.float32),
        ),
        scratch_types=[
            pltpu.VMEM((K, U), jnp.float32),
            pltpu.VMEM((K, U), jnp.float32),
            pltpu.VMEM((U,), jnp.float32),
            pltpu.VMEM((V,), jnp.int32),
            pltpu.VMEM((L,), jnp.float32),
            pltpu.SemaphoreType.DMA,
            pltpu.SemaphoreType.DMA,
        ],
    )
    def greedy(w_hbm, seq_hbm, nsz_hbm, buf0, buf1, madd, seq, szv, sem0, sem1):
        wid = lax.axis_index("s") * 2 + lax.axis_index("c")

        @pl.when(wid < B)
        def _body():
            b = wid
            lanes = lax.iota(jnp.int32, L)
            lane0 = lanes == 0
            zf = jnp.zeros((L,), jnp.float32)
            zi = jnp.zeros((L,), jnp.int32)
            for c in range(NCHUNK):
                madd[pl.ds(c * L, L)] = zf
            for c in range(V // L):
                seq[pl.ds(c * L, L)] = zi
            pltpu.async_copy(w_hbm.at[b, pl.ds(FIRST, K), :], buf0, sem0)
            pltpu.async_copy(w_hbm.at[b, pl.ds(FIRST + K, K), :], buf1, sem1)

            def do_block(g, size, buf, sem):
                pltpu.make_async_copy(
                    w_hbm.at[b, pl.ds(FIRST, K), :], buf, sem
                ).wait()

                def row(r, sz):
                    i = FIRST + g * K + r
                    NACC = 8
                    best_v = [jnp.full((L,), -jnp.inf, jnp.float32)] * NACC
                    best_i = [zi] * NACC
                    for c in range(NCHUNK):
                        a = c % NACC
                        v = buf[r, pl.ds(c * L, L)] + madd[pl.ds(c * L, L)]
                        if c == 0:
                            v = jnp.where(lane0, jnp.float32(0.0), v)
                        idx = lanes + jnp.int32(c * L)
                        upd = v > best_v[a]
                        best_i[a] = jnp.where(upd, idx, best_i[a])
                        best_v[a] = jnp.where(upd, v, best_v[a])
                    while len(best_v) > 1:
                        va, vb = best_v[0], best_v[1]
                        ia, ib = best_i[0], best_i[1]
                        take_b = (vb > va) | ((vb == va) & (ib < ia))
                        best_v = best_v[2:] + [jnp.where(take_b, vb, va)]
                        best_i = best_i[2:] + [jnp.where(take_b, ib, ia)]
                    m = jnp.max(best_v[0])
                    s = jnp.min(
                        jnp.where(best_v[0] == m, best_i[0], jnp.int32(1 << 30))
                    )
                    plsc.store_scatter(
                        seq,
                        [jnp.full((L,), i, jnp.int32)],
                        jnp.full((L,), s, jnp.int32),
                        mask=lane0,
                    )
                    plsc.store_scatter(
                        madd,
                        [jnp.full((L,), s, jnp.int32)],
                        jnp.full((L,), jnp.float32(-1e9)),
                        mask=lane0 & (s != 0),
                    )
                    return sz + m

                return lax.fori_loop(0, K, row, size)

            def pair(g2, size):
                g0 = 2 * g2
                size = do_block(g0, size, buf0, sem0)

                @pl.when(g0 + 2 < NB)
                def _():
                    pltpu.async_copy(
                        w_hbm.at[b, pl.ds(FIRST + (g0 + 2) * K, K), :],
                        buf0,
                        sem0,
                    )

                size = do_block(g0 + 1, size, buf1, sem1)

                @pl.when(g0 + 3 < NB)
                def _():
                    pltpu.async_copy(
                        w_hbm.at[b, pl.ds(FIRST + (g0 + 3) * K, K), :],
                        buf1,
                        sem1,
                    )

                return size

            size = lax.fori_loop(0, NB // 2, pair, jnp.float32(0.0))
            size = do_block(jnp.int32(NB - 1), size, buf0, sem0)
            szv[...] = jnp.full((L,), jnp.float32(0.0)) - size
            pltpu.sync_copy(seq, seq_hbm.at[b, :])
            pltpu.sync_copy(szv, nsz_hbm.at[b, :])

    seqs, nsz = greedy(weights)
    return nsz[:, 0], seqs
